# baseline (speedup 1.0000x reference)
import functools

import jax
import jax.numpy as jnp
from jax import lax
from jax.experimental import pallas as pl
from jax.experimental.pallas import tpu as pltpu
from jax.experimental.pallas import tpu_sc as plsc

NUM_VOCAB = 1000000
NUM_EMBED = 64
NUM_TOKEN = 200
BATCH = 1024

NW = 32
B_PER_W = BATCH // NW
TOK_W = B_PER_W * NUM_TOKEN
CHUNK = 128
NCHUNK = TOK_W // CHUNK
GRP = 8
LANES = 16
C_PER_ROW = NUM_EMBED // LANES
B_TOTAL = BATCH * NUM_TOKEN


def _emb_kernel(tok_hbm, table_hbm, pos_hbm, out_hbm, idxf_v, idx_v, pos_v,
                gbuf_v, rows_v, gsem):
  wid = lax.axis_index("s") * 2 + lax.axis_index("c")
  b0 = wid * B_PER_W

  pltpu.sync_copy(tok_hbm.at[pl.ds(b0, B_PER_W)], idxf_v)
  pltpu.sync_copy(pos_hbm, pos_v)

  col_offs = [LANES * c for c in range(12)] + [NUM_TOKEN - LANES]

  def conv_body(r, carry):
    base = r * NUM_TOKEN
    for off in col_offs:
      v = idxf_v[r, pl.ds(off, LANES)].astype(jnp.int32)
      idx_v[pl.ds(pl.multiple_of(base + off, 8), LANES)] = v
    return carry

  lax.fori_loop(0, B_PER_W, conv_body, 0)

  def chunk_body(j, carry):
    pltpu.async_copy(table_hbm.at[idx_v.at[pl.ds(j * CHUNK, CHUNK)]],
                     gbuf_v, gsem).wait()
    phase = lax.rem(j * CHUNK, NUM_TOKEN)

    def grp_body(g, c2):
      tg = phase + g * GRP
      for s in range(GRP):
        t = lax.rem(tg + s, NUM_TOKEN)
        for c in range(C_PER_ROW):
          sl = pl.ds(c * LANES, LANES)
          rows_v[g, s, sl] = gbuf_v[g * GRP + s, sl] + pos_v[t, sl]
      return c2

    lax.fori_loop(0, CHUNK // GRP, grp_body, 0)
    pltpu.sync_copy(
        rows_v,
        out_hbm.at[pl.ds(wid * (TOK_W // GRP) + j * (CHUNK // GRP),
                         CHUNK // GRP)])
    return carry

  lax.fori_loop(0, NCHUNK, chunk_body, 0)


@jax.jit
def _emb(tokens_f, table, positionembed):
  mesh = plsc.VectorSubcoreMesh(core_axis_name="c", subcore_axis_name="s")
  run = functools.partial(
      pl.kernel,
      mesh=mesh,
      compiler_params=pltpu.CompilerParams(use_tc_tiling_on_sc=False),
      out_type=jax.ShapeDtypeStruct((B_TOTAL // GRP, GRP, NUM_EMBED),
                                    jnp.float32),
      scratch_types=[
          pltpu.VMEM((B_PER_W, NUM_TOKEN), jnp.float32),
          pltpu.VMEM((TOK_W,), jnp.int32),
          pltpu.VMEM((NUM_TOKEN, NUM_EMBED), jnp.float32),
          pltpu.VMEM((CHUNK, NUM_EMBED), jnp.float32),
          pltpu.VMEM((CHUNK // GRP, GRP, NUM_EMBED), jnp.float32),
          pltpu.SemaphoreType.DMA,
      ],
  )(_emb_kernel)
  return run(tokens_f, table, positionembed)


def kernel(tokens, table, positionembed):
  tokens_f = tokens.astype(jnp.float32)
  out = _emb(tokens_f, table, positionembed)
  return out.reshape(BATCH, NUM_TOKEN, NUM_EMBED)
